# Initial kernel scaffold; baseline (speedup 1.0000x reference)
import jax
import jax.numpy as jnp
from jax.experimental import pallas as pl


def kernel(input0_edge_index, input0_data, input1_edge_index, input1_data, Wx1, bx1, Wx2, bx2, Wy1, by1, Wy2, by2, lx1W, lx1b, lx2W, lx2b, lx3W, lx3b, ly1W, ly1b, ly2W, ly2b, ly3W, ly3b):
    raise NotImplementedError("write your pallas kernel here")


# speedup vs baseline: 2.6404x; 2.6404x over previous
import functools

import jax
import jax.numpy as jnp
from jax.experimental import pallas as pl
from jax.experimental.pallas import tpu as pltpu

N = 10000
E = 320000
F = 128


def _mlp_body(h_ref, w1_ref, b1_ref, w2_ref, b2_ref, w3_ref, b3_ref, o_ref):
    h = h_ref[...]
    h = jax.nn.relu(jnp.dot(h, w1_ref[...], preferred_element_type=jnp.float32)
                    + b1_ref[...])
    h = jax.nn.relu(jnp.dot(h, w2_ref[...], preferred_element_type=jnp.float32)
                    + b2_ref[...])
    h = jax.nn.relu(jnp.dot(h, w3_ref[...], preferred_element_type=jnp.float32)
                    + b3_ref[...])
    o_ref[...] = h


def _mlp(h, W1, b1, W2, b2, W3, b3):
    BM = 1000
    grid = (N // BM,)
    full = lambda shape: pl.BlockSpec(shape, lambda i: (0,) * len(shape))
    return pl.pallas_call(
        _mlp_body,
        grid=grid,
        in_specs=[
            pl.BlockSpec((BM, F), lambda i: (i, 0)),
            full((F, 128)), full((128,)),
            full((128, 64)), full((64,)),
            full((64, 32)), full((32,)),
        ],
        out_specs=pl.BlockSpec((BM, 32), lambda i: (i, 0)),
        out_shape=jax.ShapeDtypeStruct((N, 32), jnp.float32),
    )(h, W1, b1, W2, b2, W3, b3)


def _outer_body(x_ref, y_ref, o_ref):
    o_ref[...] = jax.lax.dot_general(
        x_ref[...], y_ref[...],
        (((1,), (1,)), ((), ())),
        preferred_element_type=jnp.float32)


def _outer(x, y):
    BM = 400
    return pl.pallas_call(
        _outer_body,
        grid=(N // BM,),
        in_specs=[
            pl.BlockSpec((BM, 32), lambda i: (i, 0)),
            pl.BlockSpec((N, 32), lambda i: (0, 0)),
        ],
        out_specs=pl.BlockSpec((BM, N), lambda i: (i, 0)),
        out_shape=jax.ShapeDtypeStruct((N, N), jnp.float32),
    )(x, y)


def _gcn_layer(h, s, t, w, dinv, W, b):
    xp = (h @ W) * dinv[:, None]
    agg = jnp.zeros_like(xp).at[t].add(xp[s] * w[:, None]) + xp
    return jax.nn.relu(agg * dinv[:, None] + b)


def kernel(input0_edge_index, input0_data, input1_edge_index, input1_data,
           Wx1, bx1, Wx2, bx2, Wy1, by1, Wy2, by2,
           lx1W, lx1b, lx2W, lx2b, lx3W, lx3b,
           ly1W, ly1b, ly2W, ly2b, ly3W, ly3b):
    kg = jax.random.key(1)
    x_g = jax.random.normal(kg, (N, F), jnp.float32)
    x_d = jax.random.normal(jax.random.fold_in(kg, 1), (N, F), jnp.float32)

    def run_graph(x0, edge_index, data, W1, b1, W2, b2):
        s, t = edge_index[0], edge_index[1]
        w = data[s, t]
        deg = jnp.zeros((N,), jnp.float32).at[t].add(w) + 1.0
        dinv = jax.lax.rsqrt(deg)
        h = _gcn_layer(x0, s, t, w, dinv, W1, b1)
        h = _gcn_layer(h, s, t, w, dinv, W2, b2)
        h = _gcn_layer(h, s, t, w, dinv, W2, b2)
        return h

    X2 = run_graph(x_g, input1_edge_index, input1_data, Wx1, bx1, Wx2, bx2)
    Y2 = run_graph(x_d, input0_edge_index, input0_data, Wy1, by1, Wy2, by2)

    x = _mlp(X2, lx1W, lx1b, lx2W, lx2b, lx3W, lx3b)
    y = _mlp(Y2, ly1W, ly1b, ly2W, ly2b, ly3W, ly3b)
    return _outer(x, y)


# speedup vs baseline: 3.9692x; 1.5033x over previous
import functools

import jax
import jax.numpy as jnp
from jax import lax
from jax.experimental import pallas as pl
from jax.experimental.pallas import tpu as pltpu
from jax.experimental.pallas import tpu_sc as plsc

N = 10000
E = 320000
F = 128
H = 64
NC = 2
NS = 16

KP = 80
PT = E // (NC * NS)
CP = PT // KP

KA = 80
AT = E // NS
CA = AT // KA

_MESH = plsc.VectorSubcoreMesh(core_axis_name="c", subcore_axis_name="s")
_SC_PARAMS = pltpu.CompilerParams(use_tc_tiling_on_sc=False)


def _prep_body(s_hbm, t_hbm, data_hbm, ones_hbm, w_out, deg_out,
               sbuf, tbuf, ibuf, wbuf, dacc, sem):
    c = lax.axis_index("c")
    sid = lax.axis_index("s")
    wid = sid * NC + c
    pltpu.sync_copy(s_hbm.at[wid], sbuf)
    pltpu.sync_copy(t_hbm.at[wid], tbuf)

    @pl.when(sid == 0)
    def _():
        pltpu.sync_copy(ones_hbm, dacc)

    def flat_idx(i, _):
        for j in range(KP // 16):
            sl = (i, pl.ds(j * 16, 16))
            ibuf[sl] = sbuf[sl] * N + tbuf[sl]
        return 0
    lax.fori_loop(0, CP, flat_idx, 0)

    plsc.subcore_barrier()

    def chunk(i, _):
        pltpu.async_copy(data_hbm.at[ibuf.at[i]], wbuf.at[i], sem).wait()
        pltpu.sync_copy(wbuf.at[i], dacc.at[tbuf.at[i]], add=True)
        return 0
    lax.fori_loop(0, CP, chunk, 0)

    pltpu.sync_copy(wbuf, w_out.at[wid])
    plsc.subcore_barrier()

    @pl.when(sid == 0)
    def _():
        pltpu.sync_copy(dacc, deg_out.at[c])


def _sc_prep(s3, t3, data_flat, ones_n):
    return pl.kernel(
        _prep_body,
        out_type=(
            jax.ShapeDtypeStruct((NC * NS, CP, KP), jnp.float32),
            jax.ShapeDtypeStruct((NC, N), jnp.float32),
        ),
        mesh=_MESH,
        scratch_types=[
            pltpu.VMEM((CP, KP), jnp.int32),
            pltpu.VMEM((CP, KP), jnp.int32),
            pltpu.VMEM((CP, KP), jnp.int32),
            pltpu.VMEM((CP, KP), jnp.float32),
            pltpu.VMEM_SHARED((N,), jnp.float32),
            pltpu.SemaphoreType.DMA,
        ],
        compiler_params=_SC_PARAMS,
    )(s3, t3, data_flat, ones_n)


def _agg_body(s_hbm, t_hbm, w_hbm, xp_hbm, out,
              sbuf, tbuf, wbuf, rows, acc, sem):
    c = lax.axis_index("c")
    sid = lax.axis_index("s")
    pltpu.sync_copy(s_hbm.at[sid], sbuf)
    pltpu.sync_copy(t_hbm.at[sid], tbuf)
    pltpu.sync_copy(w_hbm.at[sid], wbuf)

    base = sid * 624

    @pl.when(sid < 15)
    def _():
        pltpu.sync_copy(xp_hbm.at[pl.ds(c * N + base, 624)],
                        acc.at[pl.ds(base, 624)])

    @pl.when(sid == 15)
    def _():
        pltpu.sync_copy(xp_hbm.at[pl.ds(c * N + 9360, 640)],
                        acc.at[pl.ds(9360, 640)])

    off = c * N

    def add_off(i, _):
        for j in range(KA // 16):
            sl = (i, pl.ds(j * 16, 16))
            sbuf[sl] = sbuf[sl] + off
        return 0
    lax.fori_loop(0, CA, add_off, 0)

    plsc.subcore_barrier()

    def chunk(i, _):
        pltpu.async_copy(xp_hbm.at[sbuf.at[i]], rows, sem).wait()

        def scale(g, _):
            wv = wbuf[i, pl.ds(g * 16, 16)]
            for e16 in range(16):
                e = g * 16 + e16
                w = wv[e16]
                for j in range(H // 16):
                    sl = (e, pl.ds(j * 16, 16))
                    rows[sl] = rows[sl] * w
            return 0
        lax.fori_loop(0, KA // 16, scale, 0)
        pltpu.sync_copy(rows, acc.at[tbuf.at[i]], add=True)
        return 0
    lax.fori_loop(0, CA, chunk, 0)

    plsc.subcore_barrier()

    @pl.when(sid < 15)
    def _():
        pltpu.sync_copy(acc.at[pl.ds(base, 624)],
                        out.at[c, pl.ds(base, 624)])

    @pl.when(sid == 15)
    def _():
        pltpu.sync_copy(acc.at[pl.ds(9360, 640)],
                        out.at[c, pl.ds(9360, 640)])


def _sc_agg(s3, t3, w3, xp_flat):
    return pl.kernel(
        _agg_body,
        out_type=jax.ShapeDtypeStruct((NC, N, H), jnp.float32),
        mesh=_MESH,
        scratch_types=[
            pltpu.VMEM((CA, KA), jnp.int32),
            pltpu.VMEM((CA, KA), jnp.int32),
            pltpu.VMEM((CA, KA), jnp.float32),
            pltpu.VMEM((KA, H), jnp.float32),
            pltpu.VMEM_SHARED((N, H), jnp.float32),
            pltpu.SemaphoreType.DMA,
        ],
        compiler_params=_SC_PARAMS,
    )(s3, t3, w3, xp_flat)


def _dinv(degp):
    return lax.rsqrt(degp[0] + degp[1] - 1.0)


def _first_body(h_ref, degp_ref, w_ref, o_ref):
    dinv = _dinv(degp_ref[...])
    u = jnp.dot(h_ref[...], w_ref[...], preferred_element_type=jnp.float32)
    u = u * dinv
    o_ref[0] = u[:, :H]
    o_ref[1] = u[:, H:]


def _tc_first(h, degp, W):
    BM = 1000
    return pl.pallas_call(
        _first_body,
        grid=(N // BM,),
        in_specs=[
            pl.BlockSpec((BM, F), lambda i: (i, 0)),
            pl.BlockSpec((NC, BM, 1), lambda i: (0, i, 0)),
            pl.BlockSpec((F, F), lambda i: (0, 0)),
        ],
        out_specs=pl.BlockSpec((NC, BM, H), lambda i: (0, i, 0)),
        out_shape=jax.ShapeDtypeStruct((NC, N, H), jnp.float32),
    )(h, degp, W)


def _mid_body(agg_ref, degp_ref, b_ref, w_ref, o_ref):
    dinv = _dinv(degp_ref[...])
    h = jnp.concatenate([agg_ref[0], agg_ref[1]], axis=1)
    h = jax.nn.relu(h * dinv + b_ref[...])
    u = jnp.dot(h, w_ref[...], preferred_element_type=jnp.float32)
    u = u * dinv
    o_ref[0] = u[:, :H]
    o_ref[1] = u[:, H:]


def _tc_mid(agg, degp, b, W):
    BM = 1000
    return pl.pallas_call(
        _mid_body,
        grid=(N // BM,),
        in_specs=[
            pl.BlockSpec((NC, BM, H), lambda i: (0, i, 0)),
            pl.BlockSpec((NC, BM, 1), lambda i: (0, i, 0)),
            pl.BlockSpec((1, F), lambda i: (0, 0)),
            pl.BlockSpec((F, F), lambda i: (0, 0)),
        ],
        out_specs=pl.BlockSpec((NC, BM, H), lambda i: (0, i, 0)),
        out_shape=jax.ShapeDtypeStruct((NC, N, H), jnp.float32),
    )(agg, degp, b.reshape(1, F), W)


def _mlp_body(agg_ref, degp_ref, bg_ref, w1_ref, b1_ref, w2_ref, b2_ref,
              w3_ref, b3_ref, o_ref):
    dinv = _dinv(degp_ref[...])
    h = jnp.concatenate([agg_ref[0], agg_ref[1]], axis=1)
    h = jax.nn.relu(h * dinv + bg_ref[...])
    h = jax.nn.relu(jnp.dot(h, w1_ref[...], preferred_element_type=jnp.float32)
                    + b1_ref[...])
    h = jax.nn.relu(jnp.dot(h, w2_ref[...], preferred_element_type=jnp.float32)
                    + b2_ref[...])
    h = jax.nn.relu(jnp.dot(h, w3_ref[...], preferred_element_type=jnp.float32)
                    + b3_ref[...])
    o_ref[...] = h


def _tc_mlp(agg, degp, bg, W1, b1, W2, b2, W3, b3):
    BM = 1000
    full = lambda shape: pl.BlockSpec(shape, lambda i: (0,) * len(shape))
    return pl.pallas_call(
        _mlp_body,
        grid=(N // BM,),
        in_specs=[
            pl.BlockSpec((NC, BM, H), lambda i: (0, i, 0)),
            pl.BlockSpec((NC, BM, 1), lambda i: (0, i, 0)),
            full((1, F)),
            full((F, 128)), full((128,)),
            full((128, 64)), full((64,)),
            full((64, 32)), full((32,)),
        ],
        out_specs=pl.BlockSpec((BM, 32), lambda i: (i, 0)),
        out_shape=jax.ShapeDtypeStruct((N, 32), jnp.float32),
    )(agg, degp, bg.reshape(1, F), W1, b1, W2, b2, W3, b3)


def _outer_body(x_ref, y_ref, o_ref):
    o_ref[...] = lax.dot_general(
        x_ref[...], y_ref[...],
        (((1,), (1,)), ((), ())),
        preferred_element_type=jnp.float32)


def _outer(x, y):
    BM = 400
    return pl.pallas_call(
        _outer_body,
        grid=(N // BM,),
        in_specs=[
            pl.BlockSpec((BM, 32), lambda i: (i, 0)),
            pl.BlockSpec((N, 32), lambda i: (0, 0)),
        ],
        out_specs=pl.BlockSpec((BM, N), lambda i: (i, 0)),
        out_shape=jax.ShapeDtypeStruct((N, N), jnp.float32),
    )(x, y)


def _run_graph(x0, edge_index, data, W1, b1, W2, b2,
               lW1, lb1, lW2, lb2, lW3, lb3, ones_n):
    s = edge_index[0]
    t = edge_index[1]
    s_p = s.reshape(NC * NS, CP, KP)
    t_p = t.reshape(NC * NS, CP, KP)
    w_p, degp = _sc_prep(s_p, t_p, data.reshape(D2), ones_n)
    degp3 = degp.reshape(NC, N, 1)

    s_a = s.reshape(NS, CA, KA)
    t_a = t.reshape(NS, CA, KA)
    w_a = w_p.reshape(NS, CA, KA)

    xp = _tc_first(x0, degp3, W1)
    agg = _sc_agg(s_a, t_a, w_a, xp.reshape(NC * N, H))
    xp = _tc_mid(agg, degp3, b1, W2)
    agg = _sc_agg(s_a, t_a, w_a, xp.reshape(NC * N, H))
    xp = _tc_mid(agg, degp3, b2, W2)
    agg = _sc_agg(s_a, t_a, w_a, xp.reshape(NC * N, H))
    return _tc_mlp(agg, degp3, b2, lW1, lb1, lW2, lb2, lW3, lb3)


D2 = (N * N,)


def kernel(input0_edge_index, input0_data, input1_edge_index, input1_data,
           Wx1, bx1, Wx2, bx2, Wy1, by1, Wy2, by2,
           lx1W, lx1b, lx2W, lx2b, lx3W, lx3b,
           ly1W, ly1b, ly2W, ly2b, ly3W, ly3b):
    kg = jax.random.key(1)
    x_g = jax.random.normal(kg, (N, F), jnp.float32)
    x_d = jax.random.normal(jax.random.fold_in(kg, 1), (N, F), jnp.float32)
    ones_n = jnp.ones((N,), jnp.float32)

    x = _run_graph(x_g, input1_edge_index, input1_data, Wx1, bx1, Wx2, bx2,
                   lx1W, lx1b, lx2W, lx2b, lx3W, lx3b, ones_n)
    y = _run_graph(x_d, input0_edge_index, input0_data, Wy1, by1, Wy2, by2,
                   ly1W, ly1b, ly2W, ly2b, ly3W, ly3b, ones_n)
    return _outer(x, y)


# speedup vs baseline: 6.5615x; 1.6531x over previous
import functools

import jax
import jax.numpy as jnp
from jax import lax
from jax.experimental import pallas as pl
from jax.experimental.pallas import tpu as pltpu
from jax.experimental.pallas import tpu_sc as plsc

N = 10000
E = 320000
F = 128
H = 64
NC = 2
NS = 16

KP = 80
PT = E // (NC * NS)
CP = PT // KP

KA = 80
AT = E // NS
CA = AT // KA

_MESH = plsc.VectorSubcoreMesh(core_axis_name="c", subcore_axis_name="s")
_SC_PARAMS = pltpu.CompilerParams(use_tc_tiling_on_sc=False)


def _prep_body(s_hbm, t_hbm, data_hbm, ones_hbm, w_out, deg_out,
               sbuf, tbuf, ibuf, wbuf, dacc, sem):
    c = lax.axis_index("c")
    sid = lax.axis_index("s")
    wid = sid * NC + c
    pltpu.sync_copy(s_hbm.at[wid], sbuf)
    pltpu.sync_copy(t_hbm.at[wid], tbuf)

    @pl.when(sid == 0)
    def _():
        pltpu.sync_copy(ones_hbm, dacc)

    def flat_idx(i, _):
        for j in range(KP // 16):
            sl = (i, pl.ds(j * 16, 16))
            ibuf[sl] = sbuf[sl] * N + tbuf[sl]
        return 0
    lax.fori_loop(0, CP, flat_idx, 0)

    plsc.subcore_barrier()

    def chunk(i, _):
        pltpu.async_copy(data_hbm.at[ibuf.at[i]], wbuf.at[i], sem).wait()
        pltpu.sync_copy(wbuf.at[i], dacc.at[tbuf.at[i]], add=True)
        return 0
    lax.fori_loop(0, CP, chunk, 0)

    pltpu.sync_copy(wbuf, w_out.at[wid])
    plsc.subcore_barrier()

    @pl.when(sid == 0)
    def _():
        pltpu.sync_copy(dacc, deg_out.at[c])


def _sc_prep(s3, t3, data_flat, ones_n):
    return pl.kernel(
        _prep_body,
        out_type=(
            jax.ShapeDtypeStruct((NC * NS, CP, KP), jnp.float32),
            jax.ShapeDtypeStruct((NC, N), jnp.float32),
        ),
        mesh=_MESH,
        scratch_types=[
            pltpu.VMEM((CP, KP), jnp.int32),
            pltpu.VMEM((CP, KP), jnp.int32),
            pltpu.VMEM((CP, KP), jnp.int32),
            pltpu.VMEM((CP, KP), jnp.float32),
            pltpu.VMEM_SHARED((N,), jnp.float32),
            pltpu.SemaphoreType.DMA,
        ],
        compiler_params=_SC_PARAMS,
    )(s3, t3, data_flat, ones_n)


NBUF = 5


def _agg_body(s_hbm, t_hbm, w_hbm, xp_hbm, out,
              sbuf, tbuf, wbuf, rows, acc, semg, semw):
    c = lax.axis_index("c")
    sid = lax.axis_index("s")
    pltpu.sync_copy(s_hbm.at[sid], sbuf)
    pltpu.sync_copy(t_hbm.at[sid], tbuf)
    pltpu.sync_copy(w_hbm.at[sid], wbuf)

    base = sid * 624

    @pl.when(sid < 15)
    def _():
        pltpu.sync_copy(xp_hbm.at[pl.ds(c * N + base, 624)],
                        acc.at[pl.ds(base, 624)])

    @pl.when(sid == 15)
    def _():
        pltpu.sync_copy(xp_hbm.at[pl.ds(c * N + 9360, 640)],
                        acc.at[pl.ds(9360, 640)])

    off = c * N

    def add_off(i, _):
        for j in range(KA // 16):
            sl = (i, pl.ds(j * 16, 16))
            sbuf[sl] = sbuf[sl] + off
        return 0
    lax.fori_loop(0, CA, add_off, 0)

    plsc.subcore_barrier()

    def gather_desc(i, b):
        return pltpu.make_async_copy(xp_hbm.at[sbuf.at[i]], rows.at[b],
                                     semg.at[b])

    def scatter_desc(i, b):
        return pltpu.make_async_copy(rows.at[b], acc.at[tbuf.at[i]],
                                     semw.at[b])

    for b in range(NBUF - 1):
        gather_desc(b, b).start()

    def outer(i0, _):
        for b in range(NBUF):
            i = i0 * NBUF + b
            gather_desc(i, b).wait()

            def scale(g, _):
                wv = wbuf[i, pl.ds(g * 16, 16)]
                for e16 in range(16):
                    e = g * 16 + e16
                    w = wv[e16]
                    for j in range(H // 16):
                        rows[b, e, pl.ds(j * 16, 16)] = (
                            rows[b, e, pl.ds(j * 16, 16)] * w)
                return 0
            lax.fori_loop(0, KA // 16, scale, 0)
            scatter_desc(i, b).start(add=True)

            bp = (b - 1) % NBUF

            @pl.when(i >= 1)
            def _():
                scatter_desc(i - 1, bp).wait()

            @pl.when(i + NBUF - 1 < CA)
            def _():
                gather_desc(i + NBUF - 1, bp).start()
        return 0
    lax.fori_loop(0, CA // NBUF, outer, 0)

    scatter_desc(CA - 1, (CA - 1) % NBUF).wait()

    plsc.subcore_barrier()

    @pl.when(sid < 15)
    def _():
        pltpu.sync_copy(acc.at[pl.ds(base, 624)],
                        out.at[c, pl.ds(base, 624)])

    @pl.when(sid == 15)
    def _():
        pltpu.sync_copy(acc.at[pl.ds(9360, 640)],
                        out.at[c, pl.ds(9360, 640)])


def _sc_agg(s3, t3, w3, xp_flat):
    return pl.kernel(
        _agg_body,
        out_type=jax.ShapeDtypeStruct((NC, N, H), jnp.float32),
        mesh=_MESH,
        scratch_types=[
            pltpu.VMEM((CA, KA), jnp.int32),
            pltpu.VMEM((CA, KA), jnp.int32),
            pltpu.VMEM((CA, KA), jnp.float32),
            pltpu.VMEM((NBUF, KA, H), jnp.float32),
            pltpu.VMEM_SHARED((N, H), jnp.float32),
            pltpu.SemaphoreType.DMA((NBUF,)),
            pltpu.SemaphoreType.DMA((NBUF,)),
        ],
        compiler_params=_SC_PARAMS,
    )(s3, t3, w3, xp_flat)


def _dinv(degp):
    return lax.rsqrt(degp[0] + degp[1] - 1.0)


def _first_body(h_ref, degp_ref, w_ref, o_ref):
    dinv = _dinv(degp_ref[...])
    u = jnp.dot(h_ref[...], w_ref[...], preferred_element_type=jnp.float32)
    u = u * dinv
    o_ref[0] = u[:, :H]
    o_ref[1] = u[:, H:]


def _tc_first(h, degp, W):
    BM = 1000
    return pl.pallas_call(
        _first_body,
        grid=(N // BM,),
        in_specs=[
            pl.BlockSpec((BM, F), lambda i: (i, 0)),
            pl.BlockSpec((NC, BM, 1), lambda i: (0, i, 0)),
            pl.BlockSpec((F, F), lambda i: (0, 0)),
        ],
        out_specs=pl.BlockSpec((NC, BM, H), lambda i: (0, i, 0)),
        out_shape=jax.ShapeDtypeStruct((NC, N, H), jnp.float32),
    )(h, degp, W)


def _mid_body(agg_ref, degp_ref, b_ref, w_ref, o_ref):
    dinv = _dinv(degp_ref[...])
    h = jnp.concatenate([agg_ref[0], agg_ref[1]], axis=1)
    h = jax.nn.relu(h * dinv + b_ref[...])
    u = jnp.dot(h, w_ref[...], preferred_element_type=jnp.float32)
    u = u * dinv
    o_ref[0] = u[:, :H]
    o_ref[1] = u[:, H:]


def _tc_mid(agg, degp, b, W):
    BM = 1000
    return pl.pallas_call(
        _mid_body,
        grid=(N // BM,),
        in_specs=[
            pl.BlockSpec((NC, BM, H), lambda i: (0, i, 0)),
            pl.BlockSpec((NC, BM, 1), lambda i: (0, i, 0)),
            pl.BlockSpec((1, F), lambda i: (0, 0)),
            pl.BlockSpec((F, F), lambda i: (0, 0)),
        ],
        out_specs=pl.BlockSpec((NC, BM, H), lambda i: (0, i, 0)),
        out_shape=jax.ShapeDtypeStruct((NC, N, H), jnp.float32),
    )(agg, degp, b.reshape(1, F), W)


def _mlp_body(agg_ref, degp_ref, bg_ref, w1_ref, b1_ref, w2_ref, b2_ref,
              w3_ref, b3_ref, o_ref):
    dinv = _dinv(degp_ref[...])
    h = jnp.concatenate([agg_ref[0], agg_ref[1]], axis=1)
    h = jax.nn.relu(h * dinv + bg_ref[...])
    h = jax.nn.relu(jnp.dot(h, w1_ref[...], preferred_element_type=jnp.float32)
                    + b1_ref[...])
    h = jax.nn.relu(jnp.dot(h, w2_ref[...], preferred_element_type=jnp.float32)
                    + b2_ref[...])
    h = jax.nn.relu(jnp.dot(h, w3_ref[...], preferred_element_type=jnp.float32)
                    + b3_ref[...])
    o_ref[...] = h


def _tc_mlp(agg, degp, bg, W1, b1, W2, b2, W3, b3):
    BM = 1000
    full = lambda shape: pl.BlockSpec(shape, lambda i: (0,) * len(shape))
    return pl.pallas_call(
        _mlp_body,
        grid=(N // BM,),
        in_specs=[
            pl.BlockSpec((NC, BM, H), lambda i: (0, i, 0)),
            pl.BlockSpec((NC, BM, 1), lambda i: (0, i, 0)),
            full((1, F)),
            full((F, 128)), full((128,)),
            full((128, 64)), full((64,)),
            full((64, 32)), full((32,)),
        ],
        out_specs=pl.BlockSpec((BM, 32), lambda i: (i, 0)),
        out_shape=jax.ShapeDtypeStruct((N, 32), jnp.float32),
    )(agg, degp, bg.reshape(1, F), W1, b1, W2, b2, W3, b3)


def _outer_body(x_ref, y_ref, o_ref):
    o_ref[...] = lax.dot_general(
        x_ref[...], y_ref[...],
        (((1,), (1,)), ((), ())),
        preferred_element_type=jnp.float32)


def _outer(x, y):
    BM = 400
    return pl.pallas_call(
        _outer_body,
        grid=(N // BM,),
        in_specs=[
            pl.BlockSpec((BM, 32), lambda i: (i, 0)),
            pl.BlockSpec((N, 32), lambda i: (0, 0)),
        ],
        out_specs=pl.BlockSpec((BM, N), lambda i: (i, 0)),
        out_shape=jax.ShapeDtypeStruct((N, N), jnp.float32),
    )(x, y)


def _run_graph(x0, edge_index, data, W1, b1, W2, b2,
               lW1, lb1, lW2, lb2, lW3, lb3, ones_n):
    s = edge_index[0]
    t = edge_index[1]
    s_p = s.reshape(NC * NS, CP, KP)
    t_p = t.reshape(NC * NS, CP, KP)
    w_p, degp = _sc_prep(s_p, t_p, data.reshape(D2), ones_n)
    degp3 = degp.reshape(NC, N, 1)

    s_a = s.reshape(NS, CA, KA)
    t_a = t.reshape(NS, CA, KA)
    w_a = w_p.reshape(NS, CA, KA)

    xp = _tc_first(x0, degp3, W1)
    agg = _sc_agg(s_a, t_a, w_a, xp.reshape(NC * N, H))
    xp = _tc_mid(agg, degp3, b1, W2)
    agg = _sc_agg(s_a, t_a, w_a, xp.reshape(NC * N, H))
    xp = _tc_mid(agg, degp3, b2, W2)
    agg = _sc_agg(s_a, t_a, w_a, xp.reshape(NC * N, H))
    return _tc_mlp(agg, degp3, b2, lW1, lb1, lW2, lb2, lW3, lb3)


D2 = (N * N,)


def kernel(input0_edge_index, input0_data, input1_edge_index, input1_data,
           Wx1, bx1, Wx2, bx2, Wy1, by1, Wy2, by2,
           lx1W, lx1b, lx2W, lx2b, lx3W, lx3b,
           ly1W, ly1b, ly2W, ly2b, ly3W, ly3b):
    kg = jax.random.key(1)
    x_g = jax.random.normal(kg, (N, F), jnp.float32)
    x_d = jax.random.normal(jax.random.fold_in(kg, 1), (N, F), jnp.float32)
    ones_n = jnp.ones((N,), jnp.float32)

    x = _run_graph(x_g, input1_edge_index, input1_data, Wx1, bx1, Wx2, bx2,
                   lx1W, lx1b, lx2W, lx2b, lx3W, lx3b, ones_n)
    y = _run_graph(x_d, input0_edge_index, input0_data, Wy1, by1, Wy2, by2,
                   ly1W, ly1b, ly2W, ly2b, ly3W, ly3b, ones_n)
    return _outer(x, y)


# speedup vs baseline: 6.5678x; 1.0010x over previous
import functools

import jax
import jax.numpy as jnp
from jax import lax
from jax.experimental import pallas as pl
from jax.experimental.pallas import tpu as pltpu
from jax.experimental.pallas import tpu_sc as plsc

N = 10000
E = 320000
F = 128
H = 64
NC = 2
NS = 16

KP = 80
PT = E // (NC * NS)
CP = PT // KP

KA = 80
AT = E // NS
CA = AT // KA

_MESH = plsc.VectorSubcoreMesh(core_axis_name="c", subcore_axis_name="s")
_SC_PARAMS = pltpu.CompilerParams(use_tc_tiling_on_sc=False)


def _prep_body(s_hbm, t_hbm, data_hbm, ones_hbm, w_out, deg_out,
               sbuf, tbuf, ibuf, wbuf, dacc, semg, semw):
    c = lax.axis_index("c")
    sid = lax.axis_index("s")
    wid = sid * NC + c
    pltpu.sync_copy(s_hbm.at[wid], sbuf)
    pltpu.sync_copy(t_hbm.at[wid], tbuf)

    @pl.when(sid == 0)
    def _():
        pltpu.sync_copy(ones_hbm, dacc)

    def flat_idx(i, _):
        for j in range(KP // 16):
            sl = (i, pl.ds(j * 16, 16))
            ibuf[sl] = sbuf[sl] * N + tbuf[sl]
        return 0
    lax.fori_loop(0, CP, flat_idx, 0)

    plsc.subcore_barrier()

    def gather_desc(i, b):
        return pltpu.make_async_copy(data_hbm.at[ibuf.at[i]], wbuf.at[i],
                                     semg.at[b])

    def scatter_desc(i, b):
        return pltpu.make_async_copy(wbuf.at[i], dacc.at[tbuf.at[i]],
                                     semw.at[b])

    for b in range(NBUF - 1):
        gather_desc(b, b).start()

    def outer(i0, _):
        for b in range(NBUF):
            i = i0 * NBUF + b
            gather_desc(i, b).wait()

            @pl.when(i >= NBUF)
            def _():
                scatter_desc(i - NBUF, b).wait()

            scatter_desc(i, b).start(add=True)

            @pl.when(i + NBUF - 1 < CP)
            def _():
                gather_desc(i + NBUF - 1, (b - 1) % NBUF).start()
        return 0
    lax.fori_loop(0, CP // NBUF, outer, 0)

    for b in range(NBUF):
        scatter_desc(CP - NBUF + b, b).wait()

    pltpu.sync_copy(wbuf, w_out.at[wid])
    plsc.subcore_barrier()

    @pl.when(sid == 0)
    def _():
        pltpu.sync_copy(dacc, deg_out.at[c])


def _sc_prep(s3, t3, data_flat, ones_n):
    return pl.kernel(
        _prep_body,
        out_type=(
            jax.ShapeDtypeStruct((NC * NS, CP, KP), jnp.float32),
            jax.ShapeDtypeStruct((NC, N), jnp.float32),
        ),
        mesh=_MESH,
        scratch_types=[
            pltpu.VMEM((CP, KP), jnp.int32),
            pltpu.VMEM((CP, KP), jnp.int32),
            pltpu.VMEM((CP, KP), jnp.int32),
            pltpu.VMEM((CP, KP), jnp.float32),
            pltpu.VMEM_SHARED((N,), jnp.float32),
            pltpu.SemaphoreType.DMA((NBUF,)),
            pltpu.SemaphoreType.DMA((NBUF,)),
        ],
        compiler_params=_SC_PARAMS,
    )(s3, t3, data_flat, ones_n)


NBUF = 5


def _agg_body(s_hbm, t_hbm, w_hbm, xp_hbm, out,
              sbuf, tbuf, wbuf, rows, acc, semg, semw):
    c = lax.axis_index("c")
    sid = lax.axis_index("s")
    pltpu.sync_copy(s_hbm.at[sid], sbuf)
    pltpu.sync_copy(t_hbm.at[sid], tbuf)
    pltpu.sync_copy(w_hbm.at[sid], wbuf)

    base = sid * 624

    @pl.when(sid < 15)
    def _():
        pltpu.sync_copy(xp_hbm.at[pl.ds(c * N + base, 624)],
                        acc.at[pl.ds(base, 624)])

    @pl.when(sid == 15)
    def _():
        pltpu.sync_copy(xp_hbm.at[pl.ds(c * N + 9360, 640)],
                        acc.at[pl.ds(9360, 640)])

    off = c * N

    def add_off(i, _):
        for j in range(KA // 16):
            sl = (i, pl.ds(j * 16, 16))
            sbuf[sl] = sbuf[sl] + off
        return 0
    lax.fori_loop(0, CA, add_off, 0)

    plsc.subcore_barrier()

    def gather_desc(i, b):
        return pltpu.make_async_copy(xp_hbm.at[sbuf.at[i]], rows.at[b],
                                     semg.at[b])

    def scatter_desc(i, b):
        return pltpu.make_async_copy(rows.at[b], acc.at[tbuf.at[i]],
                                     semw.at[b])

    for b in range(NBUF - 1):
        gather_desc(b, b).start()

    def outer(i0, _):
        for b in range(NBUF):
            i = i0 * NBUF + b
            gather_desc(i, b).wait()

            def scale(g, _):
                wv = wbuf[i, pl.ds(g * 16, 16)]
                for e16 in range(16):
                    e = g * 16 + e16
                    w = wv[e16]
                    for j in range(H // 16):
                        rows[b, e, pl.ds(j * 16, 16)] = (
                            rows[b, e, pl.ds(j * 16, 16)] * w)
                return 0
            lax.fori_loop(0, KA // 16, scale, 0)
            scatter_desc(i, b).start(add=True)

            bp = (b - 1) % NBUF

            @pl.when(i >= 1)
            def _():
                scatter_desc(i - 1, bp).wait()

            @pl.when(i + NBUF - 1 < CA)
            def _():
                gather_desc(i + NBUF - 1, bp).start()
        return 0
    lax.fori_loop(0, CA // NBUF, outer, 0)

    scatter_desc(CA - 1, (CA - 1) % NBUF).wait()

    plsc.subcore_barrier()

    @pl.when(sid < 15)
    def _():
        pltpu.sync_copy(acc.at[pl.ds(base, 624)],
                        out.at[c, pl.ds(base, 624)])

    @pl.when(sid == 15)
    def _():
        pltpu.sync_copy(acc.at[pl.ds(9360, 640)],
                        out.at[c, pl.ds(9360, 640)])


def _sc_agg(s3, t3, w3, xp_flat):
    return pl.kernel(
        _agg_body,
        out_type=jax.ShapeDtypeStruct((NC, N, H), jnp.float32),
        mesh=_MESH,
        scratch_types=[
            pltpu.VMEM((CA, KA), jnp.int32),
            pltpu.VMEM((CA, KA), jnp.int32),
            pltpu.VMEM((CA, KA), jnp.float32),
            pltpu.VMEM((NBUF, KA, H), jnp.float32),
            pltpu.VMEM_SHARED((N, H), jnp.float32),
            pltpu.SemaphoreType.DMA((NBUF,)),
            pltpu.SemaphoreType.DMA((NBUF,)),
        ],
        compiler_params=_SC_PARAMS,
    )(s3, t3, w3, xp_flat)


def _dinv(degp):
    return lax.rsqrt(degp[0] + degp[1] - 1.0)


def _first_body(h_ref, degp_ref, w_ref, o_ref):
    dinv = _dinv(degp_ref[...])
    u = jnp.dot(h_ref[...], w_ref[...], preferred_element_type=jnp.float32)
    u = u * dinv
    o_ref[0] = u[:, :H]
    o_ref[1] = u[:, H:]


def _tc_first(h, degp, W):
    BM = 1000
    return pl.pallas_call(
        _first_body,
        grid=(N // BM,),
        in_specs=[
            pl.BlockSpec((BM, F), lambda i: (i, 0)),
            pl.BlockSpec((NC, BM, 1), lambda i: (0, i, 0)),
            pl.BlockSpec((F, F), lambda i: (0, 0)),
        ],
        out_specs=pl.BlockSpec((NC, BM, H), lambda i: (0, i, 0)),
        out_shape=jax.ShapeDtypeStruct((NC, N, H), jnp.float32),
    )(h, degp, W)


def _mid_body(agg_ref, degp_ref, b_ref, w_ref, o_ref):
    dinv = _dinv(degp_ref[...])
    h = jnp.concatenate([agg_ref[0], agg_ref[1]], axis=1)
    h = jax.nn.relu(h * dinv + b_ref[...])
    u = jnp.dot(h, w_ref[...], preferred_element_type=jnp.float32)
    u = u * dinv
    o_ref[0] = u[:, :H]
    o_ref[1] = u[:, H:]


def _tc_mid(agg, degp, b, W):
    BM = 1000
    return pl.pallas_call(
        _mid_body,
        grid=(N // BM,),
        in_specs=[
            pl.BlockSpec((NC, BM, H), lambda i: (0, i, 0)),
            pl.BlockSpec((NC, BM, 1), lambda i: (0, i, 0)),
            pl.BlockSpec((1, F), lambda i: (0, 0)),
            pl.BlockSpec((F, F), lambda i: (0, 0)),
        ],
        out_specs=pl.BlockSpec((NC, BM, H), lambda i: (0, i, 0)),
        out_shape=jax.ShapeDtypeStruct((NC, N, H), jnp.float32),
    )(agg, degp, b.reshape(1, F), W)


def _mlp_body(agg_ref, degp_ref, bg_ref, w1_ref, b1_ref, w2_ref, b2_ref,
              w3_ref, b3_ref, o_ref):
    dinv = _dinv(degp_ref[...])
    h = jnp.concatenate([agg_ref[0], agg_ref[1]], axis=1)
    h = jax.nn.relu(h * dinv + bg_ref[...])
    h = jax.nn.relu(jnp.dot(h, w1_ref[...], preferred_element_type=jnp.float32)
                    + b1_ref[...])
    h = jax.nn.relu(jnp.dot(h, w2_ref[...], preferred_element_type=jnp.float32)
                    + b2_ref[...])
    h = jax.nn.relu(jnp.dot(h, w3_ref[...], preferred_element_type=jnp.float32)
                    + b3_ref[...])
    o_ref[...] = h


def _tc_mlp(agg, degp, bg, W1, b1, W2, b2, W3, b3):
    BM = 1000
    full = lambda shape: pl.BlockSpec(shape, lambda i: (0,) * len(shape))
    return pl.pallas_call(
        _mlp_body,
        grid=(N // BM,),
        in_specs=[
            pl.BlockSpec((NC, BM, H), lambda i: (0, i, 0)),
            pl.BlockSpec((NC, BM, 1), lambda i: (0, i, 0)),
            full((1, F)),
            full((F, 128)), full((128,)),
            full((128, 64)), full((64,)),
            full((64, 32)), full((32,)),
        ],
        out_specs=pl.BlockSpec((BM, 32), lambda i: (i, 0)),
        out_shape=jax.ShapeDtypeStruct((N, 32), jnp.float32),
    )(agg, degp, bg.reshape(1, F), W1, b1, W2, b2, W3, b3)


def _outer_body(x_ref, y_ref, o_ref):
    o_ref[...] = lax.dot_general(
        x_ref[...], y_ref[...],
        (((1,), (1,)), ((), ())),
        preferred_element_type=jnp.float32)


def _outer(x, y):
    BM = 400
    return pl.pallas_call(
        _outer_body,
        grid=(N // BM,),
        in_specs=[
            pl.BlockSpec((BM, 32), lambda i: (i, 0)),
            pl.BlockSpec((N, 32), lambda i: (0, 0)),
        ],
        out_specs=pl.BlockSpec((BM, N), lambda i: (i, 0)),
        out_shape=jax.ShapeDtypeStruct((N, N), jnp.float32),
    )(x, y)


def kernel(input0_edge_index, input0_data, input1_edge_index, input1_data,
           Wx1, bx1, Wx2, bx2, Wy1, by1, Wy2, by2,
           lx1W, lx1b, lx2W, lx2b, lx3W, lx3b,
           ly1W, ly1b, ly2W, ly2b, ly3W, ly3b):
    kg = jax.random.key(1)
    x_g = jax.random.normal(kg, (N, F), jnp.float32)
    x_d = jax.random.normal(jax.random.fold_in(kg, 1), (N, F), jnp.float32)
    ones_n = jnp.ones((N,), jnp.float32)

    def edges(edge_index):
        s, t = edge_index[0], edge_index[1]
        return ((s.reshape(NC * NS, CP, KP), t.reshape(NC * NS, CP, KP)),
                (s.reshape(NS, CA, KA), t.reshape(NS, CA, KA)))

    (sGp, tGp), (sGa, tGa) = edges(input1_edge_index)
    (sDp, tDp), (sDa, tDa) = edges(input0_edge_index)

    wG, degG = _sc_prep(sGp, tGp, input1_data.reshape(N * N), ones_n)
    wD, degD = _sc_prep(sDp, tDp, input0_data.reshape(N * N), ones_n)
    wGa = wG.reshape(NS, CA, KA)
    wDa = wD.reshape(NS, CA, KA)
    degG3 = degG.reshape(NC, N, 1)
    degD3 = degD.reshape(NC, N, 1)

    xpG = _tc_first(x_g, degG3, Wx1)
    xpD = _tc_first(x_d, degD3, Wy1)
    aggG = _sc_agg(sGa, tGa, wGa, xpG.reshape(NC * N, H))
    aggD = _sc_agg(sDa, tDa, wDa, xpD.reshape(NC * N, H))
    xpG = _tc_mid(aggG, degG3, bx1, Wx2)
    xpD = _tc_mid(aggD, degD3, by1, Wy2)
    aggG = _sc_agg(sGa, tGa, wGa, xpG.reshape(NC * N, H))
    aggD = _sc_agg(sDa, tDa, wDa, xpD.reshape(NC * N, H))
    xpG = _tc_mid(aggG, degG3, bx2, Wx2)
    xpD = _tc_mid(aggD, degD3, by2, Wy2)
    aggG = _sc_agg(sGa, tGa, wGa, xpG.reshape(NC * N, H))
    aggD = _sc_agg(sDa, tDa, wDa, xpD.reshape(NC * N, H))
    x = _tc_mlp(aggG, degG3, bx2, lx1W, lx1b, lx2W, lx2b, lx3W, lx3b)
    y = _tc_mlp(aggD, degD3, by2, ly1W, ly1b, ly2W, ly2b, ly3W, ly3b)
    return _outer(x, y)


# speedup vs baseline: 9.2918x; 1.4147x over previous
import functools

import jax
import jax.numpy as jnp
from jax import lax
from jax.experimental import pallas as pl
from jax.experimental.pallas import tpu as pltpu
from jax.experimental.pallas import tpu_sc as plsc

N = 10000
E = 320000
F = 128
H = 64
NC = 2
NS = 16

KP = 80
PT = E // (NC * NS)
CP = PT // KP

KA = 80
AT = E // NS
CA = AT // KA

_MESH = plsc.VectorSubcoreMesh(core_axis_name="c", subcore_axis_name="s")
_SC_PARAMS = pltpu.CompilerParams(use_tc_tiling_on_sc=False)


def _prep_body(s_hbm, t_hbm, data_hbm, ones_hbm, w_out, deg_out,
               sbuf, tbuf, ibuf, wbuf, dacc, semg, semw):
    c = lax.axis_index("c")
    sid = lax.axis_index("s")
    wid = sid * NC + c
    pltpu.sync_copy(s_hbm.at[wid], sbuf)
    pltpu.sync_copy(t_hbm.at[wid], tbuf)

    @pl.when(sid == 0)
    def _():
        pltpu.sync_copy(ones_hbm, dacc)

    def flat_idx(i, _):
        for j in range(KP // 16):
            sl = (i, pl.ds(j * 16, 16))
            ibuf[sl] = sbuf[sl] * N + tbuf[sl]
        return 0
    lax.fori_loop(0, CP, flat_idx, 0)

    plsc.subcore_barrier()

    def gather_desc(i, b):
        return pltpu.make_async_copy(data_hbm.at[ibuf.at[i]], wbuf.at[i],
                                     semg.at[b])

    def scatter_desc(i, b):
        return pltpu.make_async_copy(wbuf.at[i], dacc.at[tbuf.at[i]],
                                     semw.at[b])

    for b in range(NBUF - 1):
        gather_desc(b, b).start()

    def outer(i0, _):
        for b in range(NBUF):
            i = i0 * NBUF + b
            gather_desc(i, b).wait()

            @pl.when(i >= NBUF)
            def _():
                scatter_desc(i - NBUF, b).wait()

            scatter_desc(i, b).start(add=True)

            @pl.when(i + NBUF - 1 < CP)
            def _():
                gather_desc(i + NBUF - 1, (b - 1) % NBUF).start()
        return 0
    lax.fori_loop(0, CP // NBUF, outer, 0)

    for b in range(NBUF):
        scatter_desc(CP - NBUF + b, b).wait()

    pltpu.sync_copy(wbuf, w_out.at[wid])
    plsc.subcore_barrier()

    @pl.when(sid == 0)
    def _():
        pltpu.sync_copy(dacc, deg_out.at[c])


def _sc_prep(s3, t3, data_flat, ones_n):
    return pl.kernel(
        _prep_body,
        out_type=(
            jax.ShapeDtypeStruct((NC * NS, CP, KP), jnp.float32),
            jax.ShapeDtypeStruct((NC, N), jnp.float32),
        ),
        mesh=_MESH,
        scratch_types=[
            pltpu.VMEM((CP, KP), jnp.int32),
            pltpu.VMEM((CP, KP), jnp.int32),
            pltpu.VMEM((CP, KP), jnp.int32),
            pltpu.VMEM((CP, KP), jnp.float32),
            pltpu.VMEM_SHARED((N,), jnp.float32),
            pltpu.SemaphoreType.DMA((NBUF,)),
            pltpu.SemaphoreType.DMA((NBUF,)),
        ],
        compiler_params=_SC_PARAMS,
    )(s3, t3, data_flat, ones_n)


NBUF = 5


def _agg_body(s_hbm, t_hbm, w_hbm, xp_hbm, out,
              sbuf, tbuf, wbuf, rows, acc, semg, semw):
    c = lax.axis_index("c")
    sid = lax.axis_index("s")
    pltpu.sync_copy(s_hbm.at[sid], sbuf)
    pltpu.sync_copy(t_hbm.at[sid], tbuf)
    pltpu.sync_copy(w_hbm.at[sid], wbuf)

    base = sid * 624

    @pl.when(sid < 15)
    def _():
        pltpu.sync_copy(xp_hbm.at[pl.ds(c * N + base, 624)],
                        acc.at[pl.ds(base, 624)])

    @pl.when(sid == 15)
    def _():
        pltpu.sync_copy(xp_hbm.at[pl.ds(c * N + 9360, 640)],
                        acc.at[pl.ds(9360, 640)])

    off = c * N

    def add_off(i, _):
        for j in range(KA // 16):
            sl = (i, pl.ds(j * 16, 16))
            sbuf[sl] = sbuf[sl] + off
        return 0
    lax.fori_loop(0, CA, add_off, 0)

    plsc.subcore_barrier()

    def gather_desc(i, b):
        return pltpu.make_async_copy(xp_hbm.at[sbuf.at[i]], rows.at[b],
                                     semg.at[b])

    def scatter_desc(i, b):
        return pltpu.make_async_copy(rows.at[b], acc.at[tbuf.at[i]],
                                     semw.at[b])

    for b in range(NBUF - 1):
        gather_desc(b, b).start()

    def outer(i0, _):
        for b in range(NBUF):
            i = i0 * NBUF + b
            gather_desc(i, b).wait()

            for g in range(KA // 16):
                wv = wbuf[i, pl.ds(g * 16, 16)]
                for e16 in range(16):
                    e = g * 16 + e16
                    w = wv[e16]
                    for j in range(H // 16):
                        rows[b, e, pl.ds(j * 16, 16)] = (
                            rows[b, e, pl.ds(j * 16, 16)] * w)
            scatter_desc(i, b).start(add=True)

            bp = (b - 1) % NBUF

            @pl.when(i >= 1)
            def _():
                scatter_desc(i - 1, bp).wait()

            @pl.when(i + NBUF - 1 < CA)
            def _():
                gather_desc(i + NBUF - 1, bp).start()
        return 0
    lax.fori_loop(0, CA // NBUF, outer, 0)

    scatter_desc(CA - 1, (CA - 1) % NBUF).wait()

    plsc.subcore_barrier()

    @pl.when(sid < 15)
    def _():
        pltpu.sync_copy(acc.at[pl.ds(base, 624)],
                        out.at[c, pl.ds(base, 624)])

    @pl.when(sid == 15)
    def _():
        pltpu.sync_copy(acc.at[pl.ds(9360, 640)],
                        out.at[c, pl.ds(9360, 640)])


def _sc_agg(s3, t3, w3, xp_flat):
    return pl.kernel(
        _agg_body,
        out_type=jax.ShapeDtypeStruct((NC, N, H), jnp.float32),
        mesh=_MESH,
        scratch_types=[
            pltpu.VMEM((CA, KA), jnp.int32),
            pltpu.VMEM((CA, KA), jnp.int32),
            pltpu.VMEM((CA, KA), jnp.float32),
            pltpu.VMEM((NBUF, KA, H), jnp.float32),
            pltpu.VMEM_SHARED((N, H), jnp.float32),
            pltpu.SemaphoreType.DMA((NBUF,)),
            pltpu.SemaphoreType.DMA((NBUF,)),
        ],
        compiler_params=_SC_PARAMS,
    )(s3, t3, w3, xp_flat)


def _dinv(degp):
    return lax.rsqrt(degp[0] + degp[1] - 1.0)


def _first_body(h_ref, degp_ref, w_ref, o_ref):
    dinv = _dinv(degp_ref[...])
    u = jnp.dot(h_ref[...], w_ref[...], preferred_element_type=jnp.float32)
    u = u * dinv
    o_ref[0] = u[:, :H]
    o_ref[1] = u[:, H:]


def _tc_first(h, degp, W):
    BM = 1000
    return pl.pallas_call(
        _first_body,
        grid=(N // BM,),
        in_specs=[
            pl.BlockSpec((BM, F), lambda i: (i, 0)),
            pl.BlockSpec((NC, BM, 1), lambda i: (0, i, 0)),
            pl.BlockSpec((F, F), lambda i: (0, 0)),
        ],
        out_specs=pl.BlockSpec((NC, BM, H), lambda i: (0, i, 0)),
        out_shape=jax.ShapeDtypeStruct((NC, N, H), jnp.float32),
    )(h, degp, W)


def _mid_body(agg_ref, degp_ref, b_ref, w_ref, o_ref):
    dinv = _dinv(degp_ref[...])
    h = jnp.concatenate([agg_ref[0], agg_ref[1]], axis=1)
    h = jax.nn.relu(h * dinv + b_ref[...])
    u = jnp.dot(h, w_ref[...], preferred_element_type=jnp.float32)
    u = u * dinv
    o_ref[0] = u[:, :H]
    o_ref[1] = u[:, H:]


def _tc_mid(agg, degp, b, W):
    BM = 1000
    return pl.pallas_call(
        _mid_body,
        grid=(N // BM,),
        in_specs=[
            pl.BlockSpec((NC, BM, H), lambda i: (0, i, 0)),
            pl.BlockSpec((NC, BM, 1), lambda i: (0, i, 0)),
            pl.BlockSpec((1, F), lambda i: (0, 0)),
            pl.BlockSpec((F, F), lambda i: (0, 0)),
        ],
        out_specs=pl.BlockSpec((NC, BM, H), lambda i: (0, i, 0)),
        out_shape=jax.ShapeDtypeStruct((NC, N, H), jnp.float32),
    )(agg, degp, b.reshape(1, F), W)


def _mlp_body(agg_ref, degp_ref, bg_ref, w1_ref, b1_ref, w2_ref, b2_ref,
              w3_ref, b3_ref, o_ref):
    dinv = _dinv(degp_ref[...])
    h = jnp.concatenate([agg_ref[0], agg_ref[1]], axis=1)
    h = jax.nn.relu(h * dinv + bg_ref[...])
    h = jax.nn.relu(jnp.dot(h, w1_ref[...], preferred_element_type=jnp.float32)
                    + b1_ref[...])
    h = jax.nn.relu(jnp.dot(h, w2_ref[...], preferred_element_type=jnp.float32)
                    + b2_ref[...])
    h = jax.nn.relu(jnp.dot(h, w3_ref[...], preferred_element_type=jnp.float32)
                    + b3_ref[...])
    o_ref[...] = h


def _tc_mlp(agg, degp, bg, W1, b1, W2, b2, W3, b3):
    BM = 1000
    full = lambda shape: pl.BlockSpec(shape, lambda i: (0,) * len(shape))
    return pl.pallas_call(
        _mlp_body,
        grid=(N // BM,),
        in_specs=[
            pl.BlockSpec((NC, BM, H), lambda i: (0, i, 0)),
            pl.BlockSpec((NC, BM, 1), lambda i: (0, i, 0)),
            full((1, F)),
            full((F, 128)), full((128,)),
            full((128, 64)), full((64,)),
            full((64, 32)), full((32,)),
        ],
        out_specs=pl.BlockSpec((BM, 32), lambda i: (i, 0)),
        out_shape=jax.ShapeDtypeStruct((N, 32), jnp.float32),
    )(agg, degp, bg.reshape(1, F), W1, b1, W2, b2, W3, b3)


def _outer_body(x_ref, y_ref, o_ref):
    o_ref[...] = lax.dot_general(
        x_ref[...], y_ref[...],
        (((1,), (1,)), ((), ())),
        preferred_element_type=jnp.float32)


def _outer(x, y):
    BM = 400
    return pl.pallas_call(
        _outer_body,
        grid=(N // BM,),
        in_specs=[
            pl.BlockSpec((BM, 32), lambda i: (i, 0)),
            pl.BlockSpec((N, 32), lambda i: (0, 0)),
        ],
        out_specs=pl.BlockSpec((BM, N), lambda i: (i, 0)),
        out_shape=jax.ShapeDtypeStruct((N, N), jnp.float32),
    )(x, y)


def kernel(input0_edge_index, input0_data, input1_edge_index, input1_data,
           Wx1, bx1, Wx2, bx2, Wy1, by1, Wy2, by2,
           lx1W, lx1b, lx2W, lx2b, lx3W, lx3b,
           ly1W, ly1b, ly2W, ly2b, ly3W, ly3b):
    kg = jax.random.key(1)
    x_g = jax.random.normal(kg, (N, F), jnp.float32)
    x_d = jax.random.normal(jax.random.fold_in(kg, 1), (N, F), jnp.float32)
    ones_n = jnp.ones((N,), jnp.float32)

    def edges(edge_index):
        s, t = edge_index[0], edge_index[1]
        return ((s.reshape(NC * NS, CP, KP), t.reshape(NC * NS, CP, KP)),
                (s.reshape(NS, CA, KA), t.reshape(NS, CA, KA)))

    (sGp, tGp), (sGa, tGa) = edges(input1_edge_index)
    (sDp, tDp), (sDa, tDa) = edges(input0_edge_index)

    wG, degG = _sc_prep(sGp, tGp, input1_data.reshape(N * N), ones_n)
    wD, degD = _sc_prep(sDp, tDp, input0_data.reshape(N * N), ones_n)
    wGa = wG.reshape(NS, CA, KA)
    wDa = wD.reshape(NS, CA, KA)
    degG3 = degG.reshape(NC, N, 1)
    degD3 = degD.reshape(NC, N, 1)

    xpG = _tc_first(x_g, degG3, Wx1)
    xpD = _tc_first(x_d, degD3, Wy1)
    aggG = _sc_agg(sGa, tGa, wGa, xpG.reshape(NC * N, H))
    aggD = _sc_agg(sDa, tDa, wDa, xpD.reshape(NC * N, H))
    xpG = _tc_mid(aggG, degG3, bx1, Wx2)
    xpD = _tc_mid(aggD, degD3, by1, Wy2)
    aggG = _sc_agg(sGa, tGa, wGa, xpG.reshape(NC * N, H))
    aggD = _sc_agg(sDa, tDa, wDa, xpD.reshape(NC * N, H))
    xpG = _tc_mid(aggG, degG3, bx2, Wx2)
    xpD = _tc_mid(aggD, degD3, by2, Wy2)
    aggG = _sc_agg(sGa, tGa, wGa, xpG.reshape(NC * N, H))
    aggD = _sc_agg(sDa, tDa, wDa, xpD.reshape(NC * N, H))
    x = _tc_mlp(aggG, degG3, bx2, lx1W, lx1b, lx2W, lx2b, lx3W, lx3b)
    y = _tc_mlp(aggD, degD3, by2, ly1W, ly1b, ly2W, ly2b, ly3W, ly3b)
    return _outer(x, y)


# speedup vs baseline: 9.3345x; 1.0046x over previous
import functools

import jax
import jax.numpy as jnp
from jax import lax
from jax.experimental import pallas as pl
from jax.experimental.pallas import tpu as pltpu
from jax.experimental.pallas import tpu_sc as plsc

N = 10000
E = 320000
F = 128
H = 64
NC = 2
NS = 16

KP = 80
PT = E // (NC * NS)
CP = PT // KP

KA = 80
AT = E // NS
CA = AT // KA

_MESH = plsc.VectorSubcoreMesh(core_axis_name="c", subcore_axis_name="s")
_SC_PARAMS = pltpu.CompilerParams(use_tc_tiling_on_sc=False)


def _prep_body(s_hbm, t_hbm, data_hbm, ones_hbm, w_out, deg_out,
               sbuf, tbuf, ibuf, wbuf, dacc, semg, semw):
    c = lax.axis_index("c")
    sid = lax.axis_index("s")
    wid = sid * NC + c
    pltpu.sync_copy(s_hbm.at[wid], sbuf)
    pltpu.sync_copy(t_hbm.at[wid], tbuf)

    @pl.when(sid == 0)
    def _():
        pltpu.sync_copy(ones_hbm, dacc)

    def flat_idx(i, _):
        for j in range(KP // 16):
            sl = (i, pl.ds(j * 16, 16))
            ibuf[sl] = sbuf[sl] * N + tbuf[sl]
        return 0
    lax.fori_loop(0, CP, flat_idx, 0)

    plsc.subcore_barrier()

    def gather_desc(i, b):
        return pltpu.make_async_copy(data_hbm.at[ibuf.at[i]], wbuf.at[i],
                                     semg.at[b])

    def scatter_desc(i, b):
        return pltpu.make_async_copy(wbuf.at[i], dacc.at[tbuf.at[i]],
                                     semw.at[b])

    for b in range(NBUF - 1):
        gather_desc(b, b).start()

    def outer(i0, _):
        for b in range(NBUF):
            i = i0 * NBUF + b
            gather_desc(i, b).wait()

            @pl.when(i >= NBUF)
            def _():
                scatter_desc(i - NBUF, b).wait()

            scatter_desc(i, b).start(add=True)

            @pl.when(i + NBUF - 1 < CP)
            def _():
                gather_desc(i + NBUF - 1, (b - 1) % NBUF).start()
        return 0
    lax.fori_loop(0, CP // NBUF, outer, 0)

    for b in range(NBUF):
        scatter_desc(CP - NBUF + b, b).wait()

    pltpu.sync_copy(wbuf, w_out.at[wid])
    plsc.subcore_barrier()

    @pl.when(sid == 0)
    def _():
        pltpu.sync_copy(dacc, deg_out.at[c])


def _sc_prep(s3, t3, data_flat, ones_n):
    return pl.kernel(
        _prep_body,
        out_type=(
            jax.ShapeDtypeStruct((NC * NS, CP, KP), jnp.float32),
            jax.ShapeDtypeStruct((NC, N), jnp.float32),
        ),
        mesh=_MESH,
        scratch_types=[
            pltpu.VMEM((CP, KP), jnp.int32),
            pltpu.VMEM((CP, KP), jnp.int32),
            pltpu.VMEM((CP, KP), jnp.int32),
            pltpu.VMEM((CP, KP), jnp.float32),
            pltpu.VMEM_SHARED((N,), jnp.float32),
            pltpu.SemaphoreType.DMA((NBUF,)),
            pltpu.SemaphoreType.DMA((NBUF,)),
        ],
        compiler_params=_SC_PARAMS,
    )(s3, t3, data_flat, ones_n)


NBUF = 5


def _agg_body(s_hbm, t_hbm, w_hbm, xp_hbm, out,
              sbuf, tbuf, wbuf, rows, acc, semg, semw):
    c = lax.axis_index("c")
    sid = lax.axis_index("s")
    pltpu.sync_copy(s_hbm.at[sid], sbuf)
    pltpu.sync_copy(t_hbm.at[sid], tbuf)
    pltpu.sync_copy(w_hbm.at[sid], wbuf)

    base = sid * 624

    @pl.when(sid < 15)
    def _():
        pltpu.sync_copy(xp_hbm.at[pl.ds(c * N + base, 624)],
                        acc.at[pl.ds(base, 624)])

    @pl.when(sid == 15)
    def _():
        pltpu.sync_copy(xp_hbm.at[pl.ds(c * N + 9360, 640)],
                        acc.at[pl.ds(9360, 640)])

    off = c * N

    def add_off(i, _):
        for j in range(KA // 16):
            sl = (i, pl.ds(j * 16, 16))
            sbuf[sl] = sbuf[sl] + off
        return 0
    lax.fori_loop(0, CA, add_off, 0)

    plsc.subcore_barrier()

    def gather_desc(i, b):
        return pltpu.make_async_copy(xp_hbm.at[sbuf.at[i]], rows.at[b],
                                     semg.at[b])

    def scatter_desc(i, b):
        return pltpu.make_async_copy(rows.at[b], acc.at[tbuf.at[i]],
                                     semw.at[b])

    for b in range(NBUF - 1):
        gather_desc(b, b).start()

    def outer(i0, _):
        for b in range(NBUF):
            i = i0 * NBUF + b
            gather_desc(i, b).wait()

            for g in range(KA // 16):
                wv = wbuf[i, pl.ds(g * 16, 16)]
                for e16 in range(16):
                    e = g * 16 + e16
                    w = wv[e16]
                    for j in range(H // 16):
                        rows[b, e, pl.ds(j * 16, 16)] = (
                            rows[b, e, pl.ds(j * 16, 16)] * w)
            scatter_desc(i, b).start(add=True)

            bp = (b - 1) % NBUF

            @pl.when(i >= 1)
            def _():
                scatter_desc(i - 1, bp).wait()

            @pl.when(i + NBUF - 1 < CA)
            def _():
                gather_desc(i + NBUF - 1, bp).start()
        return 0
    lax.fori_loop(0, CA // NBUF, outer, 0)

    scatter_desc(CA - 1, (CA - 1) % NBUF).wait()

    plsc.subcore_barrier()

    @pl.when(sid < 15)
    def _():
        pltpu.sync_copy(acc.at[pl.ds(base, 624)],
                        out.at[c, pl.ds(base, 624)])

    @pl.when(sid == 15)
    def _():
        pltpu.sync_copy(acc.at[pl.ds(9360, 640)],
                        out.at[c, pl.ds(9360, 640)])


def _sc_agg(s3, t3, w3, xp_flat):
    return pl.kernel(
        _agg_body,
        out_type=jax.ShapeDtypeStruct((NC, N, H), jnp.float32),
        mesh=_MESH,
        scratch_types=[
            pltpu.VMEM((CA, KA), jnp.int32),
            pltpu.VMEM((CA, KA), jnp.int32),
            pltpu.VMEM((CA, KA), jnp.float32),
            pltpu.VMEM((NBUF, KA, H), jnp.float32),
            pltpu.VMEM_SHARED((N, H), jnp.float32),
            pltpu.SemaphoreType.DMA((NBUF,)),
            pltpu.SemaphoreType.DMA((NBUF,)),
        ],
        compiler_params=_SC_PARAMS,
    )(s3, t3, w3, xp_flat)


def _dinv(degp):
    return lax.rsqrt(degp[0] + degp[1] - 1.0)


def _first_body(h_ref, degp_ref, w_ref, o_ref):
    dinv = _dinv(degp_ref[...])
    u = jnp.dot(h_ref[...], w_ref[...], preferred_element_type=jnp.float32)
    u = u * dinv
    o_ref[0] = u[:, :H]
    o_ref[1] = u[:, H:]


def _tc_first(h, degp, W):
    BM = 1000
    return pl.pallas_call(
        _first_body,
        grid=(N // BM,),
        in_specs=[
            pl.BlockSpec((BM, F), lambda i: (i, 0)),
            pl.BlockSpec((NC, BM, 1), lambda i: (0, i, 0)),
            pl.BlockSpec((F, F), lambda i: (0, 0)),
        ],
        out_specs=pl.BlockSpec((NC, BM, H), lambda i: (0, i, 0)),
        out_shape=jax.ShapeDtypeStruct((NC, N, H), jnp.float32),
    )(h, degp, W)


def _mid_body(agg_ref, degp_ref, b_ref, w_ref, o_ref):
    dinv = _dinv(degp_ref[...])
    h = jnp.concatenate([agg_ref[0], agg_ref[1]], axis=1)
    h = jax.nn.relu(h * dinv + b_ref[...])
    u = jnp.dot(h, w_ref[...], preferred_element_type=jnp.float32)
    u = u * dinv
    o_ref[0] = u[:, :H]
    o_ref[1] = u[:, H:]


def _tc_mid(agg, degp, b, W):
    BM = 1000
    return pl.pallas_call(
        _mid_body,
        grid=(N // BM,),
        in_specs=[
            pl.BlockSpec((NC, BM, H), lambda i: (0, i, 0)),
            pl.BlockSpec((NC, BM, 1), lambda i: (0, i, 0)),
            pl.BlockSpec((1, F), lambda i: (0, 0)),
            pl.BlockSpec((F, F), lambda i: (0, 0)),
        ],
        out_specs=pl.BlockSpec((NC, BM, H), lambda i: (0, i, 0)),
        out_shape=jax.ShapeDtypeStruct((NC, N, H), jnp.float32),
    )(agg, degp, b.reshape(1, F), W)


def _mlp_body(agg_ref, degp_ref, bg_ref, w1_ref, b1_ref, w2_ref, b2_ref,
              w3_ref, b3_ref, o_ref):
    dinv = _dinv(degp_ref[...])
    h = jnp.concatenate([agg_ref[0], agg_ref[1]], axis=1)
    h = jax.nn.relu(h * dinv + bg_ref[...])
    h = jax.nn.relu(jnp.dot(h, w1_ref[...], preferred_element_type=jnp.float32)
                    + b1_ref[...])
    h = jax.nn.relu(jnp.dot(h, w2_ref[...], preferred_element_type=jnp.float32)
                    + b2_ref[...])
    h = jax.nn.relu(jnp.dot(h, w3_ref[...], preferred_element_type=jnp.float32)
                    + b3_ref[...])
    o_ref[...] = h


def _tc_mlp(agg, degp, bg, W1, b1, W2, b2, W3, b3):
    BM = 1000
    full = lambda shape: pl.BlockSpec(shape, lambda i: (0,) * len(shape))
    return pl.pallas_call(
        _mlp_body,
        grid=(N // BM,),
        in_specs=[
            pl.BlockSpec((NC, BM, H), lambda i: (0, i, 0)),
            pl.BlockSpec((NC, BM, 1), lambda i: (0, i, 0)),
            full((1, F)),
            full((F, 128)), full((128,)),
            full((128, 64)), full((64,)),
            full((64, 32)), full((32,)),
        ],
        out_specs=pl.BlockSpec((BM, 32), lambda i: (i, 0)),
        out_shape=jax.ShapeDtypeStruct((N, 32), jnp.float32),
    )(agg, degp, bg.reshape(1, F), W1, b1, W2, b2, W3, b3)


def _outer_body(x_ref, y_ref, o_ref):
    o_ref[...] = lax.dot_general(
        x_ref[...], y_ref[...],
        (((1,), (1,)), ((), ())),
        preferred_element_type=jnp.float32)


def _outer(x, y):
    BM = 400
    return pl.pallas_call(
        _outer_body,
        grid=(N // BM,),
        in_specs=[
            pl.BlockSpec((BM, 32), lambda i: (i, 0)),
            pl.BlockSpec((N, 32), lambda i: (0, 0)),
        ],
        out_specs=pl.BlockSpec((BM, N), lambda i: (i, 0)),
        out_shape=jax.ShapeDtypeStruct((N, N), jnp.float32),
    )(x, y)


def kernel(input0_edge_index, input0_data, input1_edge_index, input1_data,
           Wx1, bx1, Wx2, bx2, Wy1, by1, Wy2, by2,
           lx1W, lx1b, lx2W, lx2b, lx3W, lx3b,
           ly1W, ly1b, ly2W, ly2b, ly3W, ly3b):
    kg = jax.random.key(1)
    ones_n = jnp.ones((N,), jnp.float32)

    def edges(edge_index):
        s, t = edge_index[0], edge_index[1]
        return ((s.reshape(NC * NS, CP, KP), t.reshape(NC * NS, CP, KP)),
                (s.reshape(NS, CA, KA), t.reshape(NS, CA, KA)))

    (sDp, tDp), (sDa, tDa) = edges(input0_edge_index)
    (sGp, tGp), (sGa, tGa) = edges(input1_edge_index)

    x_d = jax.random.normal(jax.random.fold_in(kg, 1), (N, F), jnp.float32)
    wD, degD = _sc_prep(sDp, tDp, input0_data.reshape(N * N), ones_n)
    wDa = wD.reshape(NS, CA, KA)
    degD3 = degD.reshape(NC, N, 1)
    xpD = _tc_first(x_d, degD3, Wy1)
    aggD = _sc_agg(sDa, tDa, wDa, xpD.reshape(NC * N, H))

    x_g = jax.random.normal(kg, (N, F), jnp.float32)
    wG, degG = _sc_prep(sGp, tGp, input1_data.reshape(N * N), ones_n)
    wGa = wG.reshape(NS, CA, KA)
    degG3 = degG.reshape(NC, N, 1)
    xpG = _tc_first(x_g, degG3, Wx1)

    xpD = _tc_mid(aggD, degD3, by1, Wy2)
    aggG = _sc_agg(sGa, tGa, wGa, xpG.reshape(NC * N, H))
    aggD = _sc_agg(sDa, tDa, wDa, xpD.reshape(NC * N, H))
    xpG = _tc_mid(aggG, degG3, bx1, Wx2)
    xpD = _tc_mid(aggD, degD3, by2, Wy2)
    aggG = _sc_agg(sGa, tGa, wGa, xpG.reshape(NC * N, H))
    aggD = _sc_agg(sDa, tDa, wDa, xpD.reshape(NC * N, H))
    xpG = _tc_mid(aggG, degG3, bx2, Wx2)
    y = _tc_mlp(aggD, degD3, by2, ly1W, ly1b, ly2W, ly2b, ly3W, ly3b)
    aggG = _sc_agg(sGa, tGa, wGa, xpG.reshape(NC * N, H))
    x = _tc_mlp(aggG, degG3, bx2, lx1W, lx1b, lx2W, lx2b, lx3W, lx3b)
    return _outer(x, y)
